# Initial kernel scaffold; baseline (speedup 1.0000x reference)
import jax
import jax.numpy as jnp
from jax.experimental import pallas as pl


def kernel(input_ids, token_table, pos_table):
    raise NotImplementedError("write your pallas kernel here")


# speedup vs baseline: 1.0526x; 1.0526x over previous
import functools

import jax
import jax.numpy as jnp
from jax import lax
from jax.experimental import pallas as pl
from jax.experimental.pallas import tpu as pltpu
from jax.experimental.pallas import tpu_sc as plsc

D = 1024
BATCH = 4
SEQ = 2048
NW = 32
S_PER_W = SEQ // NW
CHUNK = 32
LANES = 16


def _emb_body(ids_hbm, tok_hbm, pos_hbm, out_hbm, idx_v, pos_v, tok_v, sem):
    c = lax.axis_index("c")
    s = lax.axis_index("s")
    wid = s * 2 + c
    seq0 = wid * S_PER_W

    pltpu.sync_copy(ids_hbm.at[wid], idx_v)
    pltpu.sync_copy(pos_hbm.at[pl.ds(seq0, S_PER_W)], pos_v)

    for b in range(BATCH):
        for ch in range(S_PER_W // CHUNK):
            row0 = b * S_PER_W + ch * CHUNK
            pltpu.async_copy(
                tok_hbm.at[idx_v.at[pl.ds(row0, CHUNK)]], tok_v, sem
            ).wait()

            def add_row(r, carry, _ch=ch):
                pos_r = _ch * CHUNK + r
                for k in range(D // LANES):
                    x = pos_v[pos_r, pl.ds(k * LANES, LANES)]
                    plsc.addupdate(tok_v.at[r, pl.ds(k * LANES, LANES)], x)
                return carry

            lax.fori_loop(0, CHUNK, add_row, 0)

            out_row = b * SEQ + seq0 + ch * CHUNK
            pltpu.sync_copy(tok_v, out_hbm.at[pl.ds(out_row, CHUNK)])


def kernel(input_ids, token_table, pos_table):
    ids = input_ids.astype(jnp.int32)
    ids_w = (
        ids.reshape(BATCH, NW, S_PER_W)
        .transpose(1, 0, 2)
        .reshape(NW, BATCH * S_PER_W)
    )
    mesh = plsc.VectorSubcoreMesh(core_axis_name="c", subcore_axis_name="s")
    run = pl.kernel(
        _emb_body,
        mesh=mesh,
        out_type=jax.ShapeDtypeStruct((BATCH * SEQ, D), jnp.float32),
        scratch_types=[
            pltpu.VMEM((BATCH * S_PER_W,), jnp.int32),
            pltpu.VMEM((S_PER_W, D), jnp.float32),
            pltpu.VMEM((CHUNK, D), jnp.float32),
            pltpu.SemaphoreType.DMA,
        ],
    )
    out = run(ids_w, token_table, pos_table)
    return out.reshape(BATCH, SEQ, D)


# speedup vs baseline: 1.4964x; 1.4216x over previous
import jax
import jax.numpy as jnp
from jax import lax
from jax.experimental import pallas as pl
from jax.experimental.pallas import tpu as pltpu
from jax.experimental.pallas import tpu_sc as plsc

D = 1024
BATCH = 4
SEQ = 2048
NW = 32
S_PER_W = SEQ // NW
CHUNK = 16
N_CH = S_PER_W // CHUNK
N_JOBS = N_CH * BATCH
LANES = 16


def _emb_body(ids_hbm, tok_hbm, pos_hbm, out_hbm, idx_v,
              p0, p1, t0, t1, t2, t3,
              ps0, ps1, gs0, gs1, gs2, gs3, ss0, ss1, ss2, ss3):
    pos_bufs = [p0, p1]
    tok_bufs = [t0, t1, t2, t3]
    psems = [ps0, ps1]
    gsems = [gs0, gs1, gs2, gs3]
    ssems = [ss0, ss1, ss2, ss3]

    c = lax.axis_index("c")
    s = lax.axis_index("s")
    wid = s * 2 + c
    seq0 = wid * S_PER_W

    pltpu.sync_copy(ids_hbm.at[wid], idx_v)

    def fire_pos(ch):
        return pltpu.async_copy(
            pos_hbm.at[pl.ds(seq0 + ch * CHUNK, CHUNK)],
            pos_bufs[ch % 2], psems[ch % 2])

    def fire_gather(j):
        return pltpu.async_copy(
            tok_hbm.at[idx_v.at[pl.ds(j * CHUNK, CHUNK)]],
            tok_bufs[j % 4], gsems[j % 4])

    def fire_store(j):
        ch, b = divmod(j, BATCH)
        row = b * SEQ + seq0 + ch * CHUNK
        return pltpu.async_copy(
            tok_bufs[j % 4], out_hbm.at[pl.ds(row, CHUNK)], ssems[j % 4])

    pend_p = {0: fire_pos(0), 1: fire_pos(1)}
    pend_g = {0: fire_gather(0), 1: fire_gather(1)}
    pend_s = {}

    for j in range(N_JOBS):
        ch, b = divmod(j, BATCH)
        if j + 2 < N_JOBS:
            if j - 2 >= 0:
                pend_s.pop(j - 2).wait()
            pend_g[j + 2] = fire_gather(j + 2)
        if b == 0:
            pend_p.pop(ch).wait()
        pend_g.pop(j).wait()

        tok = tok_bufs[j % 4]
        posb = pos_bufs[ch % 2]

        def add_row(r, carry, _tok=tok, _pos=posb):
            for k in range(D // LANES):
                x = _pos[r, pl.ds(k * LANES, LANES)]
                plsc.addupdate(_tok.at[r, pl.ds(k * LANES, LANES)], x)
            return carry

        lax.fori_loop(0, CHUNK, add_row, 0)

        if b == BATCH - 1 and ch + 2 < N_CH:
            pend_p[ch + 2] = fire_pos(ch + 2)

        pend_s[j] = fire_store(j)

    for j in sorted(pend_s):
        pend_s[j].wait()


def kernel(input_ids, token_table, pos_table):
    ids = input_ids.astype(jnp.int32)
    ids_w = (
        ids.reshape(BATCH, NW, N_CH, CHUNK)
        .transpose(1, 2, 0, 3)
        .reshape(NW, N_JOBS * CHUNK)
    )
    mesh = plsc.VectorSubcoreMesh(core_axis_name="c", subcore_axis_name="s")
    run = pl.kernel(
        _emb_body,
        mesh=mesh,
        out_type=jax.ShapeDtypeStruct((BATCH * SEQ, D), jnp.float32),
        scratch_types=(
            [pltpu.VMEM((N_JOBS * CHUNK,), jnp.int32)]
            + [pltpu.VMEM((CHUNK, D), jnp.float32)] * 2
            + [pltpu.VMEM((CHUNK, D), jnp.float32)] * 4
            + [pltpu.SemaphoreType.DMA] * 10
        ),
    )
    out = run(ids_w, token_table, pos_table)
    return out.reshape(BATCH, SEQ, D)
